# baseline (speedup 1.0000x reference)
import functools

import jax
import jax.numpy as jnp
from jax import lax
from jax.experimental import pallas as pl
from jax.experimental.pallas import tpu as pltpu
from jax.experimental.pallas import tpu_sc as plsc

GAMMA = 12.0
BATCH = 16384
HIDDEN = 128
LANES = 16

NUM_CORES = 2
NUM_SUBCORES = 16
NUM_WORKERS = NUM_CORES * NUM_SUBCORES
B_PER_W = BATCH // NUM_WORKERS
CHUNK = 64
N_CHUNKS = B_PER_W // CHUNK
GROUPS = CHUNK // LANES
DIM_CHUNKS = HIDDEN // LANES
NBUF = 4


def _make_kernel():
    mesh = plsc.VectorSubcoreMesh(core_axis_name="c", subcore_axis_name="s")

    @functools.partial(
        pl.kernel,
        mesh=mesh,
        out_type=jax.ShapeDtypeStruct((BATCH,), jnp.float32),
        compiler_params=pltpu.CompilerParams(needs_layout_passes=False,
                                             use_tc_tiling_on_sc=False),
        scratch_types=[
            pltpu.VMEM((B_PER_W, 3), jnp.int32),
            pltpu.VMEM((2 * B_PER_W,), jnp.int32),
            pltpu.VMEM((B_PER_W,), jnp.int32),
            pltpu.VMEM((NBUF, 2 * CHUNK, HIDDEN), jnp.float32),
            pltpu.VMEM((NBUF, CHUNK, HIDDEN), jnp.float32),
            pltpu.VMEM((LANES * LANES,), jnp.float32),
            pltpu.VMEM((B_PER_W,), jnp.float32),
            pltpu.SemaphoreType.DMA((NBUF,)),
            pltpu.SemaphoreType.DMA((NBUF,)),
        ],
    )
    def kge_score(sample_hbm, ent_hbm, rel_hbm, out_hbm,
                  samp, ht_idx, ridx, ht_bufs, r_bufs, p_mat, out_all,
                  sem_ht, sem_r):
        wid = lax.axis_index("s") * NUM_CORES + lax.axis_index("c")
        w_base = wid * B_PER_W
        row_ids = lax.iota(jnp.int32, LANES)
        col_h = jnp.zeros((LANES,), jnp.int32)
        col_r = jnp.full((LANES,), 1, jnp.int32)
        col_t = jnp.full((LANES,), 2, jnp.int32)

        pltpu.sync_copy(sample_hbm.at[pl.ds(w_base, B_PER_W), :], samp)
        for cc in range(N_CHUNKS):
            for i in range(GROUPS):
                rows = cc * CHUNK + i * LANES + row_ids
                h_v = plsc.load_gather(samp, [rows, col_h])
                r_v = plsc.load_gather(samp, [rows, col_r])
                t_v = plsc.load_gather(samp, [rows, col_t])
                ht_idx[pl.ds(cc * 2 * CHUNK + i * LANES, LANES)] = h_v
                ht_idx[pl.ds(cc * 2 * CHUNK + CHUNK + i * LANES, LANES)] = t_v
                ridx[pl.ds(cc * CHUNK + i * LANES, LANES)] = r_v

        def start_gathers(cc):
            b = lax.rem(cc, NBUF)
            pltpu.async_copy(
                ent_hbm.at[ht_idx.at[pl.ds(cc * 2 * CHUNK, 2 * CHUNK)]],
                ht_bufs.at[b], sem_ht.at[b])
            pltpu.async_copy(
                rel_hbm.at[ridx.at[pl.ds(cc * CHUNK, CHUNK)]],
                r_bufs.at[b], sem_r.at[b])

        def wait_gathers(b):
            pltpu.make_async_copy(
                ent_hbm.at[ht_idx.at[pl.ds(0, 2 * CHUNK)]],
                ht_bufs.at[b], sem_ht.at[b]).wait()
            pltpu.make_async_copy(
                rel_hbm.at[ridx.at[pl.ds(0, CHUNK)]],
                r_bufs.at[b], sem_r.at[b]).wait()

        for cc in range(NBUF - 1):
            start_gathers(cc)

        @pl.loop(0, N_CHUNKS)
        def _chunk(c):
            b = lax.rem(c, NBUF)

            @pl.when(c + NBUF - 1 < N_CHUNKS)
            def _():
                start_gathers(c + NBUF - 1)

            wait_gathers(b)

            for g in range(GROUPS):
                for s in range(LANES):
                    row = g * LANES + s
                    acc = None
                    for k in range(DIM_CHUNKS):
                        h = ht_bufs[b, row, pl.ds(k * LANES, LANES)]
                        t = ht_bufs[b, CHUNK + row, pl.ds(k * LANES, LANES)]
                        r = r_bufs[b, row, pl.ds(k * LANES, LANES)]
                        term = jnp.abs(h + (r - t))
                        acc = term if acc is None else acc + term
                    p_mat[pl.ds(s * LANES, LANES)] = acc
                tot = None
                for j in range(LANES):
                    col = plsc.load_gather(p_mat, [row_ids * LANES + j])
                    tot = col if tot is None else tot + col
                out_all[pl.ds(c * CHUNK + g * LANES, LANES)] = (
                    jnp.float32(GAMMA) - tot)

        pltpu.sync_copy(out_all, out_hbm.at[pl.ds(w_base, B_PER_W)])

    return kge_score


_KGE_KERNEL = _make_kernel()


def kernel(sample, entity_embedding, relation_embedding):
    scores = _KGE_KERNEL(sample, entity_embedding, relation_embedding)
    return scores.reshape(BATCH, 1)


# speedup vs baseline: 1.2105x; 1.2105x over previous
import functools

import jax
import jax.numpy as jnp
from jax import lax
from jax.experimental import pallas as pl
from jax.experimental.pallas import tpu as pltpu
from jax.experimental.pallas import tpu_sc as plsc

GAMMA = 12.0
BATCH = 16384
HIDDEN = 128
LANES = 16

NUM_CORES = 2
NUM_SUBCORES = 16
NUM_WORKERS = NUM_CORES * NUM_SUBCORES
B_PER_W = BATCH // NUM_WORKERS
CHUNK = 64
N_CHUNKS = B_PER_W // CHUNK
GROUPS = CHUNK // LANES
DIM_CHUNKS = HIDDEN // LANES
NBUF = 4


def _make_kernel():
    mesh = plsc.VectorSubcoreMesh(core_axis_name="c", subcore_axis_name="s")

    @functools.partial(
        pl.kernel,
        mesh=mesh,
        out_type=jax.ShapeDtypeStruct((BATCH,), jnp.float32),
        compiler_params=pltpu.CompilerParams(
            needs_layout_passes=False,
            disable_bounds_checks=True,
            disable_semaphore_checks=True,
            skip_device_barrier=True,
        ),
        scratch_types=[
            pltpu.VMEM((2 * B_PER_W,), jnp.int32),
            pltpu.VMEM((B_PER_W,), jnp.int32),
            pltpu.VMEM((NBUF, 2 * CHUNK, HIDDEN), jnp.float32),
            pltpu.VMEM((NBUF, CHUNK, HIDDEN), jnp.float32),
            pltpu.VMEM((LANES * LANES,), jnp.float32),
            pltpu.VMEM((B_PER_W,), jnp.float32),
            pltpu.SemaphoreType.DMA((NBUF,)),
            pltpu.SemaphoreType.DMA((NBUF,)),
        ],
    )
    def kge_score(ht_idx_hbm, r_idx_hbm, ent_hbm, rel_hbm, out_hbm,
                  ht_idx, ridx, ht_bufs, r_bufs, p_mat, out_all,
                  sem_ht, sem_r):
        wid = lax.axis_index("s") * NUM_CORES + lax.axis_index("c")
        w_base = wid * B_PER_W
        row_ids = lax.iota(jnp.int32, LANES)

        pltpu.sync_copy(ht_idx_hbm.at[pl.ds(2 * w_base, 2 * B_PER_W)], ht_idx)
        pltpu.sync_copy(r_idx_hbm.at[pl.ds(w_base, B_PER_W)], ridx)

        def start_gathers(cc):
            b = lax.rem(cc, NBUF)
            pltpu.async_copy(
                ent_hbm.at[ht_idx.at[pl.ds(cc * 2 * CHUNK, 2 * CHUNK)]],
                ht_bufs.at[b], sem_ht.at[b])
            pltpu.async_copy(
                rel_hbm.at[ridx.at[pl.ds(cc * CHUNK, CHUNK)]],
                r_bufs.at[b], sem_r.at[b])

        def wait_gathers(b):
            pltpu.make_async_copy(
                ent_hbm.at[ht_idx.at[pl.ds(0, 2 * CHUNK)]],
                ht_bufs.at[b], sem_ht.at[b]).wait()
            pltpu.make_async_copy(
                rel_hbm.at[ridx.at[pl.ds(0, CHUNK)]],
                r_bufs.at[b], sem_r.at[b]).wait()

        for cc in range(NBUF - 1):
            start_gathers(cc)

        @pl.loop(0, N_CHUNKS)
        def _chunk(c):
            b = lax.rem(c, NBUF)

            @pl.when(c + NBUF - 1 < N_CHUNKS)
            def _():
                start_gathers(c + NBUF - 1)

            wait_gathers(b)

            for g in range(GROUPS):
                for s in range(LANES):
                    row = g * LANES + s
                    acc = None
                    for k in range(DIM_CHUNKS):
                        h = ht_bufs[b, row, pl.ds(k * LANES, LANES)]
                        t = ht_bufs[b, CHUNK + row, pl.ds(k * LANES, LANES)]
                        r = r_bufs[b, row, pl.ds(k * LANES, LANES)]
                        term = jnp.abs(h + (r - t))
                        acc = term if acc is None else acc + term
                    p_mat[pl.ds(s * LANES, LANES)] = acc
                tot = None
                for j in range(LANES):
                    col = plsc.load_gather(p_mat, [row_ids * LANES + j])
                    tot = col if tot is None else tot + col
                out_all[pl.ds(c * CHUNK + g * LANES, LANES)] = (
                    jnp.float32(GAMMA) - tot)

        pltpu.sync_copy(out_all, out_hbm.at[pl.ds(w_base, B_PER_W)])

    return kge_score


_KGE_KERNEL = _make_kernel()


def kernel(sample, entity_embedding, relation_embedding):
    h_idx = sample[:, 0]
    r_idx = sample[:, 1]
    t_idx = sample[:, 2]
    ht_idx = jnp.concatenate(
        [h_idx.reshape(-1, CHUNK), t_idx.reshape(-1, CHUNK)], axis=1
    ).reshape(-1)
    scores = _KGE_KERNEL(ht_idx, r_idx, entity_embedding, relation_embedding)
    return scores.reshape(BATCH, 1)


# speedup vs baseline: 1.2163x; 1.0048x over previous
import functools

import jax
import jax.numpy as jnp
from jax import lax
from jax.experimental import pallas as pl
from jax.experimental.pallas import tpu as pltpu
from jax.experimental.pallas import tpu_sc as plsc

GAMMA = 12.0
BATCH = 16384
HIDDEN = 128
LANES = 16

NUM_CORES = 2
NUM_SUBCORES = 16
NUM_WORKERS = NUM_CORES * NUM_SUBCORES
B_PER_W = BATCH // NUM_WORKERS
CHUNK = 64
N_CHUNKS = B_PER_W // CHUNK
GROUPS = CHUNK // LANES
DIM_CHUNKS = HIDDEN // LANES
NBUF = 5


def _make_kernel():
    mesh = plsc.VectorSubcoreMesh(core_axis_name="c", subcore_axis_name="s")

    @functools.partial(
        pl.kernel,
        mesh=mesh,
        out_type=jax.ShapeDtypeStruct((BATCH,), jnp.float32),
        compiler_params=pltpu.CompilerParams(
            needs_layout_passes=False,
            disable_bounds_checks=True,
            disable_semaphore_checks=True,
            skip_device_barrier=True,
        ),
        scratch_types=[
            pltpu.VMEM((2 * B_PER_W,), jnp.int32),
            pltpu.VMEM((B_PER_W,), jnp.int32),
            pltpu.VMEM((NBUF, 2 * CHUNK, HIDDEN), jnp.float32),
            pltpu.VMEM((NBUF, CHUNK, HIDDEN), jnp.float32),
            pltpu.VMEM((LANES * LANES,), jnp.float32),
            pltpu.VMEM((B_PER_W,), jnp.float32),
            pltpu.SemaphoreType.DMA((NBUF,)),
            pltpu.SemaphoreType.DMA((NBUF,)),
        ],
    )
    def kge_score(ht_idx_hbm, r_idx_hbm, ent_hbm, rel_hbm, out_hbm,
                  ht_idx, ridx, ht_bufs, r_bufs, p_mat, out_all,
                  sem_ht, sem_r):
        wid = lax.axis_index("s") * NUM_CORES + lax.axis_index("c")
        w_base = wid * B_PER_W
        row_ids = lax.iota(jnp.int32, LANES)

        pltpu.sync_copy(ht_idx_hbm.at[pl.ds(2 * w_base, 2 * B_PER_W)], ht_idx)
        pltpu.sync_copy(r_idx_hbm.at[pl.ds(w_base, B_PER_W)], ridx)

        def start_gathers(cc):
            b = lax.rem(cc, NBUF)
            pltpu.async_copy(
                ent_hbm.at[ht_idx.at[pl.ds(cc * 2 * CHUNK, 2 * CHUNK)]],
                ht_bufs.at[b], sem_ht.at[b])
            pltpu.async_copy(
                rel_hbm.at[ridx.at[pl.ds(cc * CHUNK, CHUNK)]],
                r_bufs.at[b], sem_r.at[b])

        def wait_gathers(b):
            pltpu.make_async_copy(
                ent_hbm.at[ht_idx.at[pl.ds(0, 2 * CHUNK)]],
                ht_bufs.at[b], sem_ht.at[b]).wait()
            pltpu.make_async_copy(
                rel_hbm.at[ridx.at[pl.ds(0, CHUNK)]],
                r_bufs.at[b], sem_r.at[b]).wait()

        for cc in range(NBUF - 1):
            start_gathers(cc)

        @pl.loop(0, N_CHUNKS)
        def _chunk(c):
            b = lax.rem(c, NBUF)

            @pl.when(c + NBUF - 1 < N_CHUNKS)
            def _():
                start_gathers(c + NBUF - 1)

            wait_gathers(b)

            for g in range(GROUPS):
                for s in range(LANES):
                    row = g * LANES + s
                    acc = None
                    for k in range(DIM_CHUNKS):
                        h = ht_bufs[b, row, pl.ds(k * LANES, LANES)]
                        t = ht_bufs[b, CHUNK + row, pl.ds(k * LANES, LANES)]
                        r = r_bufs[b, row, pl.ds(k * LANES, LANES)]
                        term = jnp.abs(h + (r - t))
                        acc = term if acc is None else acc + term
                    p_mat[pl.ds(s * LANES, LANES)] = acc
                tot = None
                for j in range(LANES):
                    col = plsc.load_gather(p_mat, [row_ids * LANES + j])
                    tot = col if tot is None else tot + col
                out_all[pl.ds(c * CHUNK + g * LANES, LANES)] = (
                    jnp.float32(GAMMA) - tot)

        pltpu.sync_copy(out_all, out_hbm.at[pl.ds(w_base, B_PER_W)])

    return kge_score


_KGE_KERNEL = _make_kernel()


def kernel(sample, entity_embedding, relation_embedding):
    h_idx = sample[:, 0]
    r_idx = sample[:, 1]
    t_idx = sample[:, 2]
    ht_idx = jnp.concatenate(
        [h_idx.reshape(-1, CHUNK), t_idx.reshape(-1, CHUNK)], axis=1
    ).reshape(-1)
    scores = _KGE_KERNEL(ht_idx, r_idx, entity_embedding, relation_embedding)
    return scores.reshape(BATCH, 1)
